# Initial kernel scaffold; baseline (speedup 1.0000x reference)
import jax
import jax.numpy as jnp
from jax.experimental import pallas as pl


def kernel(x, rope, sta_mask, Wq, bq, Wk, bk, Wv, bv, gq, gk, Wo, bo):
    raise NotImplementedError("write your pallas kernel here")


# speedup vs baseline: 1.1320x; 1.1320x over previous
import math

import jax
import jax.numpy as jnp
import numpy as np
from jax.experimental import pallas as pl

_PERM_HD = np.concatenate([np.arange(0, 128, 2), np.arange(1, 128, 2)])
_PERM_C = np.concatenate([h * 128 + _PERM_HD for h in range(6)])

_B, _S, _C, _HD = 1, 4096, 768, 128
_H = _C // _HD
_S1 = _S // 64
_PTHR = 0.9
_NEG = -1e30
_BQ = 512
_BX = 512


def _qkv_body(x_ref, wq_ref, bq_ref, wk_ref, bk_ref, wv_ref, bv_ref,
              gq_ref, gk_ref, r00_ref, r01_ref, r10_ref, r11_ref,
              q_ref, k_ref, v_ref):
    x = x_ref[...]
    eps = jnp.finfo(jnp.float32).eps

    def norm_rope(t, g_ref):
        t3 = t.reshape(_BX, _H, _HD)
        ms = jnp.mean(t3 * t3, axis=-1, keepdims=True)
        t3 = t3 * jax.lax.rsqrt(ms + eps) * g_ref[...].reshape(1, 1, _HD)
        e = t3[:, :, : _HD // 2]
        o = t3[:, :, _HD // 2:]
        r00 = r00_ref[...][:, None, :]
        r01 = r01_ref[...][:, None, :]
        r10 = r10_ref[...][:, None, :]
        r11 = r11_ref[...][:, None, :]
        oe = r00 * e + r01 * o
        oo = r10 * e + r11 * o
        return jnp.concatenate([oe, oo], axis=-1).reshape(_BX, _C)

    q = jnp.dot(x, wq_ref[...], preferred_element_type=jnp.float32) + bq_ref[...]
    k = jnp.dot(x, wk_ref[...], preferred_element_type=jnp.float32) + bk_ref[...]
    v = jnp.dot(x, wv_ref[...], preferred_element_type=jnp.float32) + bv_ref[...]
    q_ref[...] = norm_rope(q, gq_ref)
    k_ref[...] = norm_rope(k, gk_ref)
    v_ref[...] = v


def _mask_body(q_ref, k_ref, sta_ref, bias_ref):
    q = q_ref[...]
    k = k_ref[...]
    qa = jnp.mean(q.reshape(_S1, 64, _HD), axis=1)
    ka = jnp.mean(k.reshape(_S1, 64, _HD), axis=1)
    logits = jax.lax.dot_general(
        qa, ka, (((1,), (1,)), ((), ())),
        preferred_element_type=jnp.float32) * (1.0 / math.sqrt(_HD))
    a = jax.nn.softmax(logits, axis=-1)
    a_j = a[:, :, None]
    a_l = a[:, None, :]
    l_idx = jax.lax.broadcasted_iota(jnp.int32, (1, _S1, _S1), 2)
    j_idx = jax.lax.broadcasted_iota(jnp.int32, (1, _S1, _S1), 1)
    le = (a_l < a_j) | ((a_l == a_j) & (l_idx <= j_idx))
    csum = jnp.sum(a_l * le.astype(jnp.float32), axis=-1)
    keep = (csum >= (1.0 - _PTHR)) | (sta_ref[0] > 0.0)
    bias_ref[0] = jnp.where(keep, 0.0, _NEG)


def _attn_body(q_ref, k_ref, v_ref, bias_ref, o_ref):
    i = pl.program_id(1)
    q = q_ref[...]
    k = k_ref[...]
    v = v_ref[...]
    s = jax.lax.dot_general(
        q, k, (((1,), (1,)), ((), ())),
        preferred_element_type=jnp.float32) * (1.0 / math.sqrt(_HD))
    nb = _BQ // 64
    rows = bias_ref[0, pl.ds(i * nb, nb), :]
    s = (s.reshape(nb, 64, _S1, 64) + rows[:, None, :, None]).reshape(_BQ, _S)
    m = jnp.max(s, axis=-1, keepdims=True)
    p = jnp.exp(s - m)
    l = jnp.sum(p, axis=-1, keepdims=True)
    o = jnp.dot(p, v, preferred_element_type=jnp.float32) / l
    o_ref[...] = o


def _proj_body(x_ref, w_ref, b_ref, o_ref):
    o_ref[...] = (jnp.dot(x_ref[...], w_ref[...],
                          preferred_element_type=jnp.float32) + b_ref[...])


def kernel(x, rope, sta_mask, Wq, bq, Wk, bk, Wv, bv, gq, gk, Wo, bo):
    f32 = jnp.float32
    x2 = x.reshape(_S, _C)
    rr = rope.reshape(_S, _HD // 2, 2, 2)
    r00, r01 = rr[:, :, 0, 0], rr[:, :, 0, 1]
    r10, r11 = rr[:, :, 1, 0], rr[:, :, 1, 1]
    staf = sta_mask.reshape(_H, _S1, _S1).astype(f32)

    full_cc = pl.BlockSpec((_C, _C), lambda i: (0, 0))
    row_c = pl.BlockSpec((1, _C), lambda i: (0, 0))
    tile_x = pl.BlockSpec((_BX, _C), lambda i: (i, 0))
    tile_r = pl.BlockSpec((_BX, _HD // 2), lambda i: (i, 0))
    g_spec = pl.BlockSpec((1, _HD), lambda i: (0, 0))

    q2, k2, v2 = pl.pallas_call(
        _qkv_body,
        grid=(_S // _BX,),
        in_specs=[tile_x, full_cc, row_c, full_cc, row_c, full_cc, row_c,
                  g_spec, g_spec, tile_r, tile_r, tile_r, tile_r],
        out_specs=[tile_x, tile_x, tile_x],
        out_shape=[jax.ShapeDtypeStruct((_S, _C), f32)] * 3,
    )(x2, Wq.T[:, _PERM_C], bq[_PERM_C].reshape(1, _C),
      Wk.T[:, _PERM_C], bk[_PERM_C].reshape(1, _C),
      Wv.T, bv.reshape(1, _C), gq[_PERM_HD].reshape(1, _HD),
      gk[_PERM_HD].reshape(1, _HD), r00, r01, r10, r11)

    head_col = pl.BlockSpec((_S, _HD), lambda h: (0, h))
    blk_spec = pl.BlockSpec((1, _S1, _S1), lambda h: (h, 0, 0))
    bias = pl.pallas_call(
        _mask_body,
        grid=(_H,),
        in_specs=[head_col, head_col, blk_spec],
        out_specs=blk_spec,
        out_shape=jax.ShapeDtypeStruct((_H, _S1, _S1), f32),
    )(q2, k2, staf)

    att = pl.pallas_call(
        _attn_body,
        grid=(_H, _S // _BQ),
        in_specs=[pl.BlockSpec((_BQ, _HD), lambda h, i: (i, h)),
                  pl.BlockSpec((_S, _HD), lambda h, i: (0, h)),
                  pl.BlockSpec((_S, _HD), lambda h, i: (0, h)),
                  pl.BlockSpec((1, _S1, _S1), lambda h, i: (h, 0, 0))],
        out_specs=pl.BlockSpec((_BQ, _HD), lambda h, i: (i, h)),
        out_shape=jax.ShapeDtypeStruct((_S, _C), f32),
    )(q2, k2, v2, bias)

    out = pl.pallas_call(
        _proj_body,
        grid=(_S // _BX,),
        in_specs=[tile_x, full_cc, row_c],
        out_specs=tile_x,
        out_shape=jax.ShapeDtypeStruct((_S, _C), f32),
    )(att, Wo.T, bo.reshape(1, _C))

    return out.reshape(_B, _S, _C)


# speedup vs baseline: 1.1392x; 1.0063x over previous
import math

import jax
import jax.numpy as jnp
import numpy as np
from jax.experimental import pallas as pl

_PERM_HD = np.concatenate([np.arange(0, 128, 2), np.arange(1, 128, 2)])
_PERM_C = np.concatenate([h * 128 + _PERM_HD for h in range(6)])

_B, _S, _C, _HD = 1, 4096, 768, 128
_H = _C // _HD
_S1 = _S // 64
_PTHR = 0.9
_NEG = -1e30
_BQ = 512
_BX = 512


def _qkv_body(x_ref, wq_ref, bq_ref, wk_ref, bk_ref, wv_ref, bv_ref,
              gq_ref, gk_ref, r00_ref, r01_ref, r10_ref, r11_ref,
              q_ref, k_ref, v_ref):
    x = x_ref[...]
    eps = jnp.finfo(jnp.float32).eps

    def norm_rope(t, g_ref):
        t3 = t.reshape(_BX, _H, _HD)
        ms = jnp.mean(t3 * t3, axis=-1, keepdims=True)
        t3 = t3 * jax.lax.rsqrt(ms + eps) * g_ref[...].reshape(1, 1, _HD)
        e = t3[:, :, : _HD // 2]
        o = t3[:, :, _HD // 2:]
        r00 = r00_ref[...][:, None, :]
        r01 = r01_ref[...][:, None, :]
        r10 = r10_ref[...][:, None, :]
        r11 = r11_ref[...][:, None, :]
        oe = r00 * e + r01 * o
        oo = r10 * e + r11 * o
        return jnp.concatenate([oe, oo], axis=-1).reshape(_BX, _C)

    xb = x.astype(jnp.bfloat16)
    q = jnp.dot(xb, wq_ref[...], preferred_element_type=jnp.float32) + bq_ref[...]
    k = jnp.dot(xb, wk_ref[...], preferred_element_type=jnp.float32) + bk_ref[...]
    v = jnp.dot(xb, wv_ref[...], preferred_element_type=jnp.float32) + bv_ref[...]
    q_ref[...] = norm_rope(q, gq_ref)
    k_ref[...] = norm_rope(k, gk_ref)
    v_ref[...] = v


def _mask_body(q_ref, k_ref, sta_ref, bias_ref):
    q = q_ref[...]
    k = k_ref[...]
    qa = jnp.mean(q.reshape(_S1, 64, _HD), axis=1)
    ka = jnp.mean(k.reshape(_S1, 64, _HD), axis=1)
    logits = jax.lax.dot_general(
        qa, ka, (((1,), (1,)), ((), ())),
        preferred_element_type=jnp.float32) * (1.0 / math.sqrt(_HD))
    a = jax.nn.softmax(logits, axis=-1)
    a_j = a[:, :, None]
    a_l = a[:, None, :]
    l_idx = jax.lax.broadcasted_iota(jnp.int32, (1, _S1, _S1), 2)
    j_idx = jax.lax.broadcasted_iota(jnp.int32, (1, _S1, _S1), 1)
    le = (a_l < a_j) | ((a_l == a_j) & (l_idx <= j_idx))
    csum = jnp.sum(a_l * le.astype(jnp.float32), axis=-1)
    keep = (csum >= (1.0 - _PTHR)) | (sta_ref[0] > 0.0)
    bias_ref[0] = jnp.where(keep, 0.0, _NEG)


def _attn_body(q_ref, k_ref, v_ref, bias_ref, o_ref):
    i = pl.program_id(1)
    q = q_ref[...].astype(jnp.bfloat16)
    k = k_ref[...].astype(jnp.bfloat16)
    v = v_ref[...].astype(jnp.bfloat16)
    s = jax.lax.dot_general(
        q, k, (((1,), (1,)), ((), ())),
        preferred_element_type=jnp.float32) * (1.0 / math.sqrt(_HD))
    nb = _BQ // 64
    rows = bias_ref[0, pl.ds(i * nb, nb), :]
    s = (s.reshape(nb, 64, _S1, 64) + rows[:, None, :, None]).reshape(_BQ, _S)
    m = jnp.max(s, axis=-1, keepdims=True)
    p = jnp.exp(s - m)
    l = jnp.sum(p, axis=-1, keepdims=True)
    o = jnp.dot(p.astype(jnp.bfloat16), v,
                preferred_element_type=jnp.float32) / l
    o_ref[...] = o


def _proj_body(x_ref, w_ref, b_ref, o_ref):
    o_ref[...] = (jnp.dot(x_ref[...].astype(jnp.bfloat16), w_ref[...],
                          preferred_element_type=jnp.float32) + b_ref[...])


def kernel(x, rope, sta_mask, Wq, bq, Wk, bk, Wv, bv, gq, gk, Wo, bo):
    f32 = jnp.float32
    x2 = x.reshape(_S, _C)
    rr = rope.reshape(_S, _HD // 2, 2, 2)
    r00, r01 = rr[:, :, 0, 0], rr[:, :, 0, 1]
    r10, r11 = rr[:, :, 1, 0], rr[:, :, 1, 1]
    staf = sta_mask.reshape(_H, _S1, _S1).astype(f32)

    full_cc = pl.BlockSpec((_C, _C), lambda i: (0, 0))
    row_c = pl.BlockSpec((1, _C), lambda i: (0, 0))
    tile_x = pl.BlockSpec((_BX, _C), lambda i: (i, 0))
    tile_r = pl.BlockSpec((_BX, _HD // 2), lambda i: (i, 0))
    g_spec = pl.BlockSpec((1, _HD), lambda i: (0, 0))

    q2, k2, v2 = pl.pallas_call(
        _qkv_body,
        grid=(_S // _BX,),
        in_specs=[tile_x, full_cc, row_c, full_cc, row_c, full_cc, row_c,
                  g_spec, g_spec, tile_r, tile_r, tile_r, tile_r],
        out_specs=[tile_x, tile_x, tile_x],
        out_shape=[jax.ShapeDtypeStruct((_S, _C), f32)] * 3,
    )(x2, Wq.T[:, _PERM_C].astype(jnp.bfloat16), bq[_PERM_C].reshape(1, _C),
      Wk.T[:, _PERM_C].astype(jnp.bfloat16), bk[_PERM_C].reshape(1, _C),
      Wv.T.astype(jnp.bfloat16), bv.reshape(1, _C),
      gq[_PERM_HD].reshape(1, _HD), gk[_PERM_HD].reshape(1, _HD),
      r00, r01, r10, r11)

    head_col = pl.BlockSpec((_S, _HD), lambda h: (0, h))
    blk_spec = pl.BlockSpec((1, _S1, _S1), lambda h: (h, 0, 0))
    bias = pl.pallas_call(
        _mask_body,
        grid=(_H,),
        in_specs=[head_col, head_col, blk_spec],
        out_specs=blk_spec,
        out_shape=jax.ShapeDtypeStruct((_H, _S1, _S1), f32),
    )(q2, k2, staf)

    att = pl.pallas_call(
        _attn_body,
        grid=(_H, _S // _BQ),
        in_specs=[pl.BlockSpec((_BQ, _HD), lambda h, i: (i, h)),
                  pl.BlockSpec((_S, _HD), lambda h, i: (0, h)),
                  pl.BlockSpec((_S, _HD), lambda h, i: (0, h)),
                  pl.BlockSpec((1, _S1, _S1), lambda h, i: (h, 0, 0))],
        out_specs=pl.BlockSpec((_BQ, _HD), lambda h, i: (i, h)),
        out_shape=jax.ShapeDtypeStruct((_S, _C), f32),
    )(q2, k2, v2, bias)

    out = pl.pallas_call(
        _proj_body,
        grid=(_S // _BX,),
        in_specs=[tile_x, full_cc, row_c],
        out_specs=tile_x,
        out_shape=jax.ShapeDtypeStruct((_S, _C), f32),
    )(att, Wo.T.astype(jnp.bfloat16), bo.reshape(1, _C))

    return out.reshape(_B, _S, _C)


# speedup vs baseline: 1.9542x; 1.7154x over previous
import math

import jax
import jax.numpy as jnp
import numpy as np
from jax.experimental import pallas as pl

_PERM_HD = np.concatenate([np.arange(0, 128, 2), np.arange(1, 128, 2)])
_PERM_C = np.concatenate([h * 128 + _PERM_HD for h in range(6)])

_B, _S, _C, _HD = 1, 4096, 768, 128
_H = _C // _HD
_S1 = _S // 64
_PTHR = 0.9
_NEG = -1e30
_BQ = 512
_BX = 512


def _qkv_body(x_ref, wq_ref, bq_ref, wk_ref, bk_ref, wv_ref, bv_ref,
              gq_ref, gk_ref, r00_ref, r01_ref, r10_ref, r11_ref,
              q_ref, k_ref, v_ref):
    x = x_ref[...]
    eps = jnp.finfo(jnp.float32).eps

    def norm_rope(t, g_ref):
        t3 = t.reshape(_BX, _H, _HD)
        ms = jnp.mean(t3 * t3, axis=-1, keepdims=True)
        t3 = t3 * jax.lax.rsqrt(ms + eps) * g_ref[...].reshape(1, 1, _HD)
        e = t3[:, :, : _HD // 2]
        o = t3[:, :, _HD // 2:]
        r00 = r00_ref[...][:, None, :]
        r01 = r01_ref[...][:, None, :]
        r10 = r10_ref[...][:, None, :]
        r11 = r11_ref[...][:, None, :]
        oe = r00 * e + r01 * o
        oo = r10 * e + r11 * o
        return jnp.concatenate([oe, oo], axis=-1).reshape(_BX, _C)

    xb = x.astype(jnp.bfloat16)
    q = jnp.dot(xb, wq_ref[...], preferred_element_type=jnp.float32) + bq_ref[...]
    k = jnp.dot(xb, wk_ref[...], preferred_element_type=jnp.float32) + bk_ref[...]
    v = jnp.dot(xb, wv_ref[...], preferred_element_type=jnp.float32) + bv_ref[...]
    q_ref[...] = norm_rope(q, gq_ref)
    k_ref[...] = norm_rope(k, gk_ref)
    v_ref[...] = v


def _mask_body(q_ref, k_ref, sta_ref, bias_ref):
    q = q_ref[...]
    k = k_ref[...]
    qa = jnp.mean(q.reshape(_S1, 64, _HD), axis=1)
    ka = jnp.mean(k.reshape(_S1, 64, _HD), axis=1)
    logits = jax.lax.dot_general(
        qa, ka, (((1,), (1,)), ((), ())),
        preferred_element_type=jnp.float32) * (1.0 / math.sqrt(_HD))
    a = jax.nn.softmax(logits, axis=-1)
    a_j = a[:, :, None]
    a_l = a[:, None, :]
    l_idx = jax.lax.broadcasted_iota(jnp.int32, (1, _S1, _S1), 2)
    j_idx = jax.lax.broadcasted_iota(jnp.int32, (1, _S1, _S1), 1)
    le = (a_l < a_j) | ((a_l == a_j) & (l_idx <= j_idx))
    csum = jnp.sum(a_l * le.astype(jnp.float32), axis=-1)
    keep = (csum >= (1.0 - _PTHR)) | (sta_ref[0] > 0.0)
    bcol = jnp.where(keep, 0.0, _NEG).astype(jnp.bfloat16).T
    bias_ref[0] = jnp.broadcast_to(
        bcol[:, None, :], (_S1, 64, _S1)).reshape(_S, _S1)


def _attn_body(q_ref, k_ref, v_ref, bias_ref, o_ref):
    i = pl.program_id(1)
    nb = _BQ // 64
    q = q_ref[...].astype(jnp.bfloat16)
    k = k_ref[...].astype(jnp.bfloat16)
    v = v_ref[...].astype(jnp.bfloat16)
    r_blk = jax.lax.broadcasted_iota(jnp.int32, (_BQ, _S1), 0) // 64 + i * nb
    c_idx = jax.lax.broadcasted_iota(jnp.int32, (_BQ, _S1), 1)
    onehot = (c_idx == r_blk).astype(jnp.bfloat16)
    q_aug = jnp.concatenate([q, onehot], axis=1)
    k_aug = jnp.concatenate([k, bias_ref[0]], axis=1)
    s = jax.lax.dot_general(
        q_aug, k_aug, (((1,), (1,)), ((), ())),
        preferred_element_type=jnp.float32) * (1.0 / math.sqrt(_HD))
    m = jnp.max(s, axis=-1, keepdims=True)
    p = jnp.exp(s - m)
    l = jnp.sum(p, axis=-1, keepdims=True)
    o = jnp.dot(p.astype(jnp.bfloat16), v,
                preferred_element_type=jnp.float32) / l
    o_ref[...] = o


def _proj_body(x_ref, w_ref, b_ref, o_ref):
    o_ref[...] = (jnp.dot(x_ref[...].astype(jnp.bfloat16), w_ref[...],
                          preferred_element_type=jnp.float32) + b_ref[...])


def kernel(x, rope, sta_mask, Wq, bq, Wk, bk, Wv, bv, gq, gk, Wo, bo):
    f32 = jnp.float32
    x2 = x.reshape(_S, _C)
    rr = rope.reshape(_S, _HD // 2, 2, 2)
    r00, r01 = rr[:, :, 0, 0], rr[:, :, 0, 1]
    r10, r11 = rr[:, :, 1, 0], rr[:, :, 1, 1]
    staf = sta_mask.reshape(_H, _S1, _S1).astype(f32)

    full_cc = pl.BlockSpec((_C, _C), lambda i: (0, 0))
    row_c = pl.BlockSpec((1, _C), lambda i: (0, 0))
    tile_x = pl.BlockSpec((_BX, _C), lambda i: (i, 0))
    tile_r = pl.BlockSpec((_BX, _HD // 2), lambda i: (i, 0))
    g_spec = pl.BlockSpec((1, _HD), lambda i: (0, 0))

    q2, k2, v2 = pl.pallas_call(
        _qkv_body,
        grid=(_S // _BX,),
        in_specs=[tile_x, full_cc, row_c, full_cc, row_c, full_cc, row_c,
                  g_spec, g_spec, tile_r, tile_r, tile_r, tile_r],
        out_specs=[tile_x, tile_x, tile_x],
        out_shape=[jax.ShapeDtypeStruct((_S, _C), f32)] * 3,
    )(x2, Wq.T[:, _PERM_C].astype(jnp.bfloat16), bq[_PERM_C].reshape(1, _C),
      Wk.T[:, _PERM_C].astype(jnp.bfloat16), bk[_PERM_C].reshape(1, _C),
      Wv.T.astype(jnp.bfloat16), bv.reshape(1, _C),
      gq[_PERM_HD].reshape(1, _HD), gk[_PERM_HD].reshape(1, _HD),
      r00, r01, r10, r11)

    head_col = pl.BlockSpec((_S, _HD), lambda h: (0, h))
    sta_spec = pl.BlockSpec((1, _S1, _S1), lambda h: (h, 0, 0))
    bias_spec = pl.BlockSpec((1, _S, _S1), lambda h: (h, 0, 0))
    bias = pl.pallas_call(
        _mask_body,
        grid=(_H,),
        in_specs=[head_col, head_col, sta_spec],
        out_specs=bias_spec,
        out_shape=jax.ShapeDtypeStruct((_H, _S, _S1), jnp.bfloat16),
    )(q2, k2, staf)

    att = pl.pallas_call(
        _attn_body,
        grid=(_H, _S // _BQ),
        in_specs=[pl.BlockSpec((_BQ, _HD), lambda h, i: (i, h)),
                  pl.BlockSpec((_S, _HD), lambda h, i: (0, h)),
                  pl.BlockSpec((_S, _HD), lambda h, i: (0, h)),
                  pl.BlockSpec((1, _S, _S1), lambda h, i: (h, 0, 0))],
        out_specs=pl.BlockSpec((_BQ, _HD), lambda h, i: (i, h)),
        out_shape=jax.ShapeDtypeStruct((_S, _C), f32),
    )(q2, k2, v2, bias)

    out = pl.pallas_call(
        _proj_body,
        grid=(_S // _BX,),
        in_specs=[tile_x, full_cc, row_c],
        out_specs=tile_x,
        out_shape=jax.ShapeDtypeStruct((_S, _C), f32),
    )(att, Wo.T.astype(jnp.bfloat16), bo.reshape(1, _C))

    return out.reshape(_B, _S, _C)


# speedup vs baseline: 2.0491x; 1.0486x over previous
import math

import jax
import jax.numpy as jnp
import numpy as np
from jax.experimental import pallas as pl

_PERM_HD = np.concatenate([np.arange(0, 128, 2), np.arange(1, 128, 2)])
_PERM_C = np.concatenate([h * 128 + _PERM_HD for h in range(6)])

_B, _S, _C, _HD = 1, 4096, 768, 128
_H = _C // _HD
_S1 = _S // 64
_PTHR = 0.9
_NEG = -1e30
_BQ = 1024
_BX = 512


def _qkv_body(x_ref, wq_ref, bq_ref, wk_ref, bk_ref, wv_ref, bv_ref,
              gq_ref, gk_ref, r00_ref, r01_ref, r10_ref, r11_ref,
              q_ref, k_ref, v_ref):
    x = x_ref[...]
    eps = jnp.finfo(jnp.float32).eps

    def norm_rope(t, g_ref):
        t3 = t.reshape(_BX, _H, _HD)
        ms = jnp.mean(t3 * t3, axis=-1, keepdims=True)
        t3 = t3 * jax.lax.rsqrt(ms + eps) * g_ref[...].reshape(1, 1, _HD)
        e = t3[:, :, : _HD // 2]
        o = t3[:, :, _HD // 2:]
        r00 = r00_ref[...][:, None, :]
        r01 = r01_ref[...][:, None, :]
        r10 = r10_ref[...][:, None, :]
        r11 = r11_ref[...][:, None, :]
        oe = r00 * e + r01 * o
        oo = r10 * e + r11 * o
        return jnp.concatenate([oe, oo], axis=-1).reshape(_BX, _C)

    xb = x.astype(jnp.bfloat16)
    q = jnp.dot(xb, wq_ref[...], preferred_element_type=jnp.float32) + bq_ref[...]
    k = jnp.dot(xb, wk_ref[...], preferred_element_type=jnp.float32) + bk_ref[...]
    v = jnp.dot(xb, wv_ref[...], preferred_element_type=jnp.float32) + bv_ref[...]
    q_ref[...] = norm_rope(q, gq_ref)
    k_ref[...] = norm_rope(k, gk_ref)
    v_ref[...] = v


def _mask_body(q_ref, k_ref, sta_ref, bias_ref):
    q = q_ref[...]
    k = k_ref[...]
    qa = jnp.mean(q.reshape(_S1, 64, _HD), axis=1)
    ka = jnp.mean(k.reshape(_S1, 64, _HD), axis=1)
    logits = jax.lax.dot_general(
        qa, ka, (((1,), (1,)), ((), ())),
        preferred_element_type=jnp.float32) * (1.0 / math.sqrt(_HD))
    a = jax.nn.softmax(logits, axis=-1)
    a_j = a[:, :, None]
    a_l = a[:, None, :]
    l_idx = jax.lax.broadcasted_iota(jnp.int32, (1, _S1, _S1), 2)
    j_idx = jax.lax.broadcasted_iota(jnp.int32, (1, _S1, _S1), 1)
    le = (a_l < a_j) | ((a_l == a_j) & (l_idx <= j_idx))
    csum = jnp.sum(a_l * le.astype(jnp.float32), axis=-1)
    keep = (csum >= (1.0 - _PTHR)) | (sta_ref[0] > 0.0)
    bcol = jnp.where(keep, 0.0, _NEG).astype(jnp.bfloat16).T
    bias_ref[0] = jnp.broadcast_to(
        bcol[:, None, :], (_S1, 64, _S1)).reshape(_S, _S1)


def _attn_body(q_ref, k_ref, v_ref, bias_ref, o_ref):
    i = pl.program_id(1)
    nb = _BQ // 64
    q = q_ref[...].astype(jnp.bfloat16)
    k = k_ref[...].astype(jnp.bfloat16)
    v = v_ref[...].astype(jnp.bfloat16)
    r_blk = jax.lax.broadcasted_iota(jnp.int32, (_BQ, _S1), 0) // 64 + i * nb
    c_idx = jax.lax.broadcasted_iota(jnp.int32, (_BQ, _S1), 1)
    onehot = (c_idx == r_blk).astype(jnp.bfloat16)
    q_aug = jnp.concatenate([q, onehot], axis=1)
    k_aug = jnp.concatenate([k, bias_ref[0]], axis=1)
    s = jax.lax.dot_general(
        q_aug, k_aug, (((1,), (1,)), ((), ())),
        preferred_element_type=jnp.float32)
    m = jnp.max(s, axis=-1, keepdims=True)
    p = jnp.exp((s - m) * (1.0 / math.sqrt(_HD)))
    l = jnp.sum(p, axis=-1, keepdims=True)
    o = jnp.dot(p.astype(jnp.bfloat16), v,
                preferred_element_type=jnp.float32) / l
    o_ref[...] = o


def _proj_body(x_ref, w_ref, b_ref, o_ref):
    o_ref[...] = (jnp.dot(x_ref[...].astype(jnp.bfloat16), w_ref[...],
                          preferred_element_type=jnp.float32) + b_ref[...])


def kernel(x, rope, sta_mask, Wq, bq, Wk, bk, Wv, bv, gq, gk, Wo, bo):
    f32 = jnp.float32
    x2 = x.reshape(_S, _C)
    rr = rope.reshape(_S, _HD // 2, 2, 2)
    r00, r01 = rr[:, :, 0, 0], rr[:, :, 0, 1]
    r10, r11 = rr[:, :, 1, 0], rr[:, :, 1, 1]
    staf = sta_mask.reshape(_H, _S1, _S1).astype(f32)

    full_cc = pl.BlockSpec((_C, _C), lambda i: (0, 0))
    row_c = pl.BlockSpec((1, _C), lambda i: (0, 0))
    tile_x = pl.BlockSpec((_BX, _C), lambda i: (i, 0))
    tile_r = pl.BlockSpec((_BX, _HD // 2), lambda i: (i, 0))
    g_spec = pl.BlockSpec((1, _HD), lambda i: (0, 0))

    q2, k2, v2 = pl.pallas_call(
        _qkv_body,
        grid=(_S // _BX,),
        in_specs=[tile_x, full_cc, row_c, full_cc, row_c, full_cc, row_c,
                  g_spec, g_spec, tile_r, tile_r, tile_r, tile_r],
        out_specs=[tile_x, tile_x, tile_x],
        out_shape=[jax.ShapeDtypeStruct((_S, _C), f32)] * 3,
    )(x2, Wq.T[:, _PERM_C].astype(jnp.bfloat16), bq[_PERM_C].reshape(1, _C),
      Wk.T[:, _PERM_C].astype(jnp.bfloat16), bk[_PERM_C].reshape(1, _C),
      Wv.T.astype(jnp.bfloat16), bv.reshape(1, _C),
      gq[_PERM_HD].reshape(1, _HD), gk[_PERM_HD].reshape(1, _HD),
      r00, r01, r10, r11)

    head_col = pl.BlockSpec((_S, _HD), lambda h: (0, h))
    sta_spec = pl.BlockSpec((1, _S1, _S1), lambda h: (h, 0, 0))
    bias_spec = pl.BlockSpec((1, _S, _S1), lambda h: (h, 0, 0))
    bias = pl.pallas_call(
        _mask_body,
        grid=(_H,),
        in_specs=[head_col, head_col, sta_spec],
        out_specs=bias_spec,
        out_shape=jax.ShapeDtypeStruct((_H, _S, _S1), jnp.bfloat16),
    )(q2, k2, staf)

    att = pl.pallas_call(
        _attn_body,
        grid=(_H, _S // _BQ),
        in_specs=[pl.BlockSpec((_BQ, _HD), lambda h, i: (i, h)),
                  pl.BlockSpec((_S, _HD), lambda h, i: (0, h)),
                  pl.BlockSpec((_S, _HD), lambda h, i: (0, h)),
                  pl.BlockSpec((1, _S, _S1), lambda h, i: (h, 0, 0))],
        out_specs=pl.BlockSpec((_BQ, _HD), lambda h, i: (i, h)),
        out_shape=jax.ShapeDtypeStruct((_S, _C), f32),
    )(q2, k2, v2, bias)

    out = pl.pallas_call(
        _proj_body,
        grid=(_S // _BX,),
        in_specs=[tile_x, full_cc, row_c],
        out_specs=tile_x,
        out_shape=jax.ShapeDtypeStruct((_S, _C), f32),
    )(att, Wo.T.astype(jnp.bfloat16), bo.reshape(1, _C))

    return out.reshape(_B, _S, _C)
